# baseline (speedup 1.0000x reference)
import functools

import jax
import jax.numpy as jnp
from jax import lax
from jax.experimental import pallas as pl
from jax.experimental.pallas import tpu as pltpu
from jax.experimental.pallas import tpu_sc as plsc

L = 16
NW = 32
CHUNK = 128

_PERM_DNUMS = lax.GatherDimensionNumbers(
    offset_dims=(), collapsed_slice_dims=(0,), start_index_map=(0,))


def _permute(x, idx):
    return lax.gather(x, idx[:, None], _PERM_DNUMS, slice_sizes=(1,),
                      mode=lax.GatherScatterMode.PROMISE_IN_BOUNDS)


@functools.lru_cache(maxsize=None)
def _make_kernel(B: int, F: int):
    assert F == 2 * L
    b_per_w = B // NW
    n_chunks = b_per_w // CHUNK
    mesh = plsc.VectorSubcoreMesh(core_axis_name="c", subcore_axis_name="s")

    @functools.partial(
        pl.kernel,
        mesh=mesh,
        out_type=jax.ShapeDtypeStruct((B,), jnp.float32),
        compiler_params=pltpu.CompilerParams(use_tc_tiling_on_sc=False),
        scratch_types=[
            pltpu.VMEM((n_chunks, CHUNK), jnp.int32),
            pltpu.VMEM((n_chunks, CHUNK), jnp.int32),
            pltpu.VMEM((b_per_w, F), jnp.float32),
            pltpu.VMEM((b_per_w, F), jnp.float32),
            pltpu.VMEM((b_per_w,), jnp.float32),
            pltpu.SemaphoreType.DMA,
        ],
    )
    def pmf_kernel(user_hbm, item_hbm, uids_hbm, iids_hbm, out_hbm,
                   uidx, iidx, urows, vrows, outv, sem):
        wid = lax.axis_index("s") * 2 + lax.axis_index("c")
        base = wid * b_per_w
        pltpu.sync_copy(uids_hbm.at[wid], uidx)
        pltpu.sync_copy(iids_hbm.at[wid], iidx)
        copies = []
        for j in range(n_chunks):
            copies.append(pltpu.async_copy(
                user_hbm.at[uidx.at[j]], urows.at[pl.ds(j * CHUNK, CHUNK)], sem))
            copies.append(pltpu.async_copy(
                item_hbm.at[iidx.at[j]], vrows.at[pl.ds(j * CHUNK, CHUNK)], sem))
        for c in copies:
            c.wait()

        lanes = lax.iota(jnp.int32, L)
        perms = [lanes ^ s for s in (1, 2, 4, 8)]
        masks = [(lanes & s) == 0 for s in (1, 2, 4, 8)]

        def block(it, carry):
            r0 = it * L
            vecs = []
            for i in range(L):
                r = r0 + i
                u0 = urows[r, 0:L]
                u1 = urows[r, L:2 * L]
                v0 = vrows[r, 0:L]
                v1 = vrows[r, L:2 * L]
                vecs.append(u0 * v0 + u1 * v1)
            for lev in range(4):
                nxt = []
                for p in range(0, len(vecs), 2):
                    x, y = vecs[p], vecs[p + 1]
                    px = _permute(x, perms[lev])
                    py = _permute(y, perms[lev])
                    nxt.append(jnp.where(masks[lev], x + px, y + py))
                vecs = nxt
            outv[pl.ds(r0, L)] = vecs[0]
            return carry

        lax.fori_loop(0, b_per_w // L, block, 0)
        pltpu.sync_copy(outv, out_hbm.at[pl.ds(base, b_per_w)])

    return pmf_kernel


def kernel(user_table, item_table, user_ids, item_ids):
    B = user_ids.shape[0]
    F = user_table.shape[1]
    k = _make_kernel(B, F)
    uids = user_ids.astype(jnp.int32).reshape(NW, -1, CHUNK)
    iids = item_ids.astype(jnp.int32).reshape(NW, -1, CHUNK)
    return k(user_table, item_table, uids, iids)


# speedup vs baseline: 1.4017x; 1.4017x over previous
import functools

import jax
import jax.numpy as jnp
from jax import lax
from jax.experimental import pallas as pl
from jax.experimental.pallas import tpu as pltpu
from jax.experimental.pallas import tpu_sc as plsc

L = 16
NW = 32

_PERM_DNUMS = lax.GatherDimensionNumbers(
    offset_dims=(), collapsed_slice_dims=(0,), start_index_map=(0,))


def _permute(x, idx):
    return lax.gather(x, idx[:, None], _PERM_DNUMS, slice_sizes=(1,),
                      mode=lax.GatherScatterMode.PROMISE_IN_BOUNDS)


@functools.lru_cache(maxsize=None)
def _make_kernel(B: int, F: int):
    assert F == 2 * L
    b_per_w = B // NW
    n_groups = b_per_w // L
    mesh = plsc.VectorSubcoreMesh(core_axis_name="c", subcore_axis_name="s")

    @functools.partial(
        pl.kernel,
        mesh=mesh,
        out_type=jax.ShapeDtypeStruct((B,), jnp.float32),
        scratch_types=[
            pltpu.VMEM((b_per_w,), jnp.int32),
            pltpu.VMEM((b_per_w,), jnp.int32),
            pltpu.VMEM((2 * L, 8, F), jnp.float32),
            pltpu.VMEM((2 * L, 8, F), jnp.float32),
            pltpu.VMEM((b_per_w,), jnp.float32),
            pltpu.SemaphoreType.DMA,
            pltpu.SemaphoreType.DMA,
        ],
    )
    def pmf_kernel(user_hbm, item_hbm, uids_hbm, iids_hbm, out_hbm,
                   uraw, iraw, gu, gi, outv, sem_u, sem_v):
        wid = lax.axis_index("s") * 2 + lax.axis_index("c")
        base = wid * b_per_w
        pltpu.sync_copy(uids_hbm.at[pl.ds(base, b_per_w)], uraw)
        pltpu.sync_copy(iids_hbm.at[pl.ds(base, b_per_w)], iraw)

        lanes = lax.iota(jnp.int32, L)
        perms = [lanes ^ s for s in (1, 2, 4, 8)]
        masks = [(lanes & s) == 0 for s in (1, 2, 4, 8)]

        def fire(g):
            o = g * L
            slot = (g % 2) * L
            idu = jnp.bitwise_and(uraw[pl.ds(o, L)], ~7)
            idv = jnp.bitwise_and(iraw[pl.ds(o, L)], ~7)
            for j in range(L):
                pltpu.async_copy(
                    user_hbm.at[pl.ds(pl.multiple_of(idu[j], 8), 8), :],
                    gu.at[slot + j], sem_u)
                pltpu.async_copy(
                    item_hbm.at[pl.ds(pl.multiple_of(idv[j], 8), 8), :],
                    gi.at[slot + j], sem_v)

        def drain_and_compute(g):
            for _ in range(L):
                pltpu.make_async_copy(
                    user_hbm.at[pl.ds(0, 8), :], gu.at[0], sem_u).wait()
                pltpu.make_async_copy(
                    item_hbm.at[pl.ds(0, 8), :], gi.at[0], sem_v).wait()
            o = g * L
            slot = (g % 2) * L
            su = jnp.bitwise_and(uraw[pl.ds(o, L)], 7)
            si = jnp.bitwise_and(iraw[pl.ds(o, L)], 7)
            vecs = []
            for j in range(L):
                u0 = gu[slot + j, su[j], 0:L]
                u1 = gu[slot + j, su[j], L:2 * L]
                v0 = gi[slot + j, si[j], 0:L]
                v1 = gi[slot + j, si[j], L:2 * L]
                vecs.append(u0 * v0 + u1 * v1)
            for lev in range(4):
                nxt = []
                for p in range(0, len(vecs), 2):
                    x, y = vecs[p], vecs[p + 1]
                    px = _permute(x, perms[lev])
                    py = _permute(y, perms[lev])
                    nxt.append(jnp.where(masks[lev], x + px, y + py))
                vecs = nxt
            outv[pl.ds(o, L)] = vecs[0]

        def step(g, carry):
            pl.when(g < n_groups)(lambda: fire(g))
            pl.when(g >= 1)(lambda: drain_and_compute(g - 1))
            return carry

        lax.fori_loop(0, n_groups + 1, step, 0)
        pltpu.sync_copy(outv, out_hbm.at[pl.ds(base, b_per_w)])

    return pmf_kernel


def kernel(user_table, item_table, user_ids, item_ids):
    B = user_ids.shape[0]
    F = user_table.shape[1]
    k = _make_kernel(B, F)
    return k(user_table, item_table,
             user_ids.astype(jnp.int32), item_ids.astype(jnp.int32))


# speedup vs baseline: 3.8103x; 2.7184x over previous
import functools

import jax
import jax.numpy as jnp
from jax import lax
from jax.experimental import pallas as pl
from jax.experimental.pallas import tpu as pltpu
from jax.experimental.pallas import tpu_sc as plsc

L = 16
NW = 32
NSLOT = 8
SUB = 4

_PERM_DNUMS = lax.GatherDimensionNumbers(
    offset_dims=(), collapsed_slice_dims=(0,), start_index_map=(0,))


def _permute(x, idx):
    return lax.gather(x, idx[:, None], _PERM_DNUMS, slice_sizes=(1,),
                      mode=lax.GatherScatterMode.PROMISE_IN_BOUNDS)


@functools.lru_cache(maxsize=None)
def _make_kernel(B: int, F: int):
    assert F == 2 * L
    b_per_w = B // NW
    n_groups = b_per_w // L
    mesh = plsc.VectorSubcoreMesh(core_axis_name="c", subcore_axis_name="s")

    @functools.partial(
        pl.kernel,
        mesh=mesh,
        out_type=jax.ShapeDtypeStruct((B,), jnp.float32),
        compiler_params=pltpu.CompilerParams(needs_layout_passes=False),
        scratch_types=[
            pltpu.VMEM((b_per_w,), jnp.int32),
            pltpu.VMEM((b_per_w,), jnp.int32),
            pltpu.VMEM((NSLOT, F, 128), jnp.float32),
            pltpu.VMEM((NSLOT, F, 128), jnp.float32),
            pltpu.VMEM((b_per_w,), jnp.float32),
            pltpu.SemaphoreType.DMA,
            pltpu.SemaphoreType.DMA,
        ],
    )
    def pmf_kernel(user_hbm, item_hbm, uids_hbm, iids_hbm, out_hbm,
                   uraw, iraw, gu, gi, outv, sem_u, sem_v):
        wid = lax.axis_index("s") * 2 + lax.axis_index("c")
        base = wid * b_per_w
        pltpu.sync_copy(uids_hbm.at[pl.ds(base, b_per_w)], uraw)
        pltpu.sync_copy(iids_hbm.at[pl.ds(base, b_per_w)], iraw)

        lanes = lax.iota(jnp.int32, L)
        rows_lo = lanes
        rows_hi = lanes + L
        perms = [lanes ^ s for s in (1, 2, 4, 8)]
        masks = [(lanes & s) == 0 for s in (1, 2, 4, 8)]

        def group_body(g, carry):
            o = g * L
            idu = uraw[pl.ds(o, L)]
            idv = iraw[pl.ds(o, L)]
            cu = jnp.bitwise_and(idu, ~127)
            cv = jnp.bitwise_and(idv, ~127)
            lu = jnp.bitwise_and(idu, 127)
            lv = jnp.bitwise_and(idv, 127)

            def fire(k):
                for j in range(SUB):
                    i = k * SUB + j
                    s = i % NSLOT
                    pltpu.async_copy(
                        user_hbm.at[:, pl.ds(pl.multiple_of(cu[i], 128), 128)],
                        gu.at[s], sem_u)
                    pltpu.async_copy(
                        item_hbm.at[:, pl.ds(pl.multiple_of(cv[i], 128), 128)],
                        gi.at[s], sem_v)

            def drain_and_extract(k, vecs):
                for _ in range(SUB):
                    pltpu.make_async_copy(
                        user_hbm.at[:, pl.ds(0, 128)], gu.at[0], sem_u).wait()
                    pltpu.make_async_copy(
                        item_hbm.at[:, pl.ds(0, 128)], gi.at[0], sem_v).wait()
                for j in range(SUB):
                    i = k * SUB + j
                    s = i % NSLOT
                    lu_b = jnp.full((L,), lu[i], jnp.int32)
                    lv_b = jnp.full((L,), lv[i], jnp.int32)
                    u0 = plsc.load_gather(gu.at[s], [rows_lo, lu_b])
                    u1 = plsc.load_gather(gu.at[s], [rows_hi, lu_b])
                    v0 = plsc.load_gather(gi.at[s], [rows_lo, lv_b])
                    v1 = plsc.load_gather(gi.at[s], [rows_hi, lv_b])
                    vecs.append(u0 * v0 + u1 * v1)
                return vecs

            n_sub = L // SUB
            vecs = []
            fire(0)
            for k in range(1, n_sub):
                fire(k)
                vecs = drain_and_extract(k - 1, vecs)
            vecs = drain_and_extract(n_sub - 1, vecs)

            for lev in range(4):
                nxt = []
                for p in range(0, len(vecs), 2):
                    x, y = vecs[p], vecs[p + 1]
                    px = _permute(x, perms[lev])
                    py = _permute(y, perms[lev])
                    nxt.append(jnp.where(masks[lev], x + px, y + py))
                vecs = nxt
            outv[pl.ds(o, L)] = vecs[0]
            return carry

        lax.fori_loop(0, n_groups, group_body, 0)
        pltpu.sync_copy(outv, out_hbm.at[pl.ds(base, b_per_w)])

    return pmf_kernel


def kernel(user_table, item_table, user_ids, item_ids):
    B = user_ids.shape[0]
    F = user_table.shape[1]
    k = _make_kernel(B, F)
    return k(user_table.T, item_table.T,
             user_ids.astype(jnp.int32), item_ids.astype(jnp.int32))
